# speedup vs baseline: 1.6610x; 1.6610x over previous
import functools

import jax
import jax.numpy as jnp
from jax.experimental import pallas as pl

INPUT_DIM = 256
NUM_HEADS = 4
HEAD_DIM = INPUT_DIM // NUM_HEADS


def _beta_kernel(xe_ref, e_ref, beta_ref):
    beta_ref[...] = jnp.dot(xe_ref[...], e_ref[...],
                            preferred_element_type=jnp.float32)


def _main_kernel(xq_ref, wq_ref, g_ref, s_ref, bg_ref, xqs_ref, ex_ref):
    xq = jnp.dot(xq_ref[...], wq_ref[...], preferred_element_type=jnp.float32)
    gamma = jnp.dot(xq, g_ref[...], preferred_element_type=jnp.float32)
    alpha = bg_ref[...] + gamma
    alpha = jnp.where(alpha >= 0, alpha, 0.01 * alpha)
    ex = jnp.exp(alpha)
    ex_ref[...] = ex
    exr = jnp.dot(ex, s_ref[...], preferred_element_type=jnp.float32)
    xqs_ref[...] = xq * exr


def kernel(x_q, x_edge, edge_ids, w_q, weight):
    num_q = x_q.shape[0]
    num_e = x_edge.shape[0]
    eid = edge_ids.astype(jnp.int32)

    hsel = (jnp.arange(INPUT_DIM) // HEAD_DIM)[:, None] == jnp.arange(NUM_HEADS)[None, :]
    E = jnp.where(hsel, weight[:, :HEAD_DIM].reshape(INPUT_DIM)[:, None], 0.0)
    G = jnp.where(hsel, weight[:, HEAD_DIM:].reshape(INPUT_DIM)[:, None], 0.0)
    S = hsel.T.astype(jnp.float32)

    bm_e = 640
    beta = pl.pallas_call(
        _beta_kernel,
        grid=(num_e // bm_e,),
        in_specs=[
            pl.BlockSpec((bm_e, INPUT_DIM), lambda i: (i, 0)),
            pl.BlockSpec((INPUT_DIM, NUM_HEADS), lambda i: (0, 0)),
        ],
        out_specs=pl.BlockSpec((bm_e, NUM_HEADS), lambda i: (i, 0)),
        out_shape=jax.ShapeDtypeStruct((num_e, NUM_HEADS), jnp.float32),
    )(x_edge, E)

    bg = jnp.take(beta, eid, axis=0)

    bm = 512
    xqs, ex = pl.pallas_call(
        _main_kernel,
        grid=(num_q // bm,),
        in_specs=[
            pl.BlockSpec((bm, INPUT_DIM), lambda i: (i, 0)),
            pl.BlockSpec((INPUT_DIM, INPUT_DIM), lambda i: (0, 0)),
            pl.BlockSpec((INPUT_DIM, NUM_HEADS), lambda i: (0, 0)),
            pl.BlockSpec((NUM_HEADS, INPUT_DIM), lambda i: (0, 0)),
            pl.BlockSpec((bm, NUM_HEADS), lambda i: (i, 0)),
        ],
        out_specs=[
            pl.BlockSpec((bm, INPUT_DIM), lambda i: (i, 0)),
            pl.BlockSpec((bm, NUM_HEADS), lambda i: (i, 0)),
        ],
        out_shape=[
            jax.ShapeDtypeStruct((num_q, INPUT_DIM), jnp.float32),
            jax.ShapeDtypeStruct((num_q, NUM_HEADS), jnp.float32),
        ],
    )(x_q, w_q, G, S, bg)

    numer = jax.ops.segment_sum(xqs, eid, num_segments=num_e)
    denom = jax.ops.segment_sum(ex, eid, num_segments=num_e)
    dr = jnp.repeat(denom, HEAD_DIM, axis=1)
    return numer / (dr + 1e-16)
